# baseline (speedup 1.0000x reference)
import functools

import jax
import jax.numpy as jnp
from jax import lax
from jax.experimental import pallas as pl
from jax.experimental.pallas import tpu as pltpu
from jax.experimental.pallas import tpu_sc as plsc

N_NODE = 10000
D_FEAT = 128
D_OUT = 16
N_EDGES = 320000

NC, NS = 2, 16
CHUNK = 2048
EDGES_PER_TILE = 20480
E_PAD = EDGES_PER_TILE * NS
N_CHUNKS = EDGES_PER_TILE // CHUNK
ROWS_PER_TILE = 624
LAST_ROWS = N_NODE - ROWS_PER_TILE * (NS - 1)
ACC_ROWS = 10240
ZROWS = ACC_ROWS // NS
BLK = 2000


def _stage1_body(xa_ref, xt_ref, wl1_ref, bl1_ref, wr1_ref, wl2_ref, bl2_ref,
                 wr2_ref, h1_ref, h2_ref, r1_ref, r2_ref):
    xa = xa_ref[...]
    xt = xt_ref[...]
    h1_ref[...] = jnp.dot(xa, wl1_ref[...], preferred_element_type=jnp.float32)
    h2_ref[...] = jnp.dot(xt, wl2_ref[...], preferred_element_type=jnp.float32)
    r1_ref[...] = jnp.dot(xt, wr1_ref[...],
                          preferred_element_type=jnp.float32) + bl1_ref[...]
    r2_ref[...] = jnp.dot(xa, wr2_ref[...],
                          preferred_element_type=jnp.float32) + bl2_ref[...]


def _stage3_body(acc1_ref, cnt1_ref, r1_ref, acc2_ref, cnt2_ref, r2_ref,
                 out_t_ref, out_a_ref):
    out_t_ref[...] = (acc1_ref[...] / jnp.maximum(cnt1_ref[...], 1.0)
                      + r1_ref[...])
    out_a_ref[...] = (acc2_ref[...] / jnp.maximum(cnt2_ref[...], 1.0)
                      + r2_ref[...])


def _segment_kernel(h1_hbm, src1_hbm, dst1_hbm, h2_hbm, src2_hbm, dst2_hbm,
                    out1_hbm, cnt1_hbm, out2_hbm, cnt2_hbm,
                    acc, cnt, h_sp, src_all, dst_all, rows_v0, rows_v1, ones_v,
                    zero_c, g_sem0, g_sem1, g2_sem0, g2_sem1, s_sem0, s_sem1,
                    c_sem0, c_sem1):
    cid = lax.axis_index("c")
    sid = lax.axis_index("s")

    z16 = jnp.zeros((16,), jnp.float32)
    o16 = jnp.ones((16,), jnp.float32)

    def zbody(i, _):
        rows_v0[i, pl.ds(0, 16)] = z16
        return 0
    lax.fori_loop(0, ZROWS, zbody, 0)

    def zcbody(i, _):
        zero_c[pl.ds(i * 16, 16)] = z16
        return 0
    lax.fori_loop(0, ZROWS // 16, zcbody, 0)

    def obody(i, _):
        ones_v[pl.ds(i * 16, 16)] = o16
        return 0
    lax.fori_loop(0, CHUNK // 16, obody, 0)

    pltpu.sync_copy(rows_v0.at[pl.ds(0, ZROWS)], acc.at[pl.ds(sid * ZROWS, ZROWS)])
    pltpu.sync_copy(zero_c, cnt.at[pl.ds(sid * ZROWS, ZROWS)])
    plsc.subcore_barrier()

    def process(h_hbm, src_hbm, dst_hbm, out_hbm, cnt_hbm):
        base = sid * EDGES_PER_TILE
        pltpu.sync_copy(src_hbm.at[pl.ds(base, EDGES_PER_TILE)], src_all)
        pltpu.sync_copy(dst_hbm.at[pl.ds(base, EDGES_PER_TILE)], dst_all)

        @pl.when(sid < NS - 1)
        def _():
            pltpu.sync_copy(h_hbm.at[pl.ds(sid * ROWS_PER_TILE, ROWS_PER_TILE)],
                            h_sp.at[pl.ds(sid * ROWS_PER_TILE, ROWS_PER_TILE)])

        @pl.when(sid == NS - 1)
        def _():
            pltpu.sync_copy(h_hbm.at[pl.ds((NS - 1) * ROWS_PER_TILE, LAST_ROWS)],
                            h_sp.at[pl.ds((NS - 1) * ROWS_PER_TILE, LAST_ROWS)])
        plsc.subcore_barrier()

        rows = (rows_v0, rows_v1)
        g_sem = (g_sem0, g_sem1)
        g2_sem = (g2_sem0, g2_sem1)
        s_sem = (s_sem0, s_sem1)
        c_sem = (c_sem0, c_sem1)
        HALF = CHUNK // 2

        def gather_start(g, b):
            pltpu.async_copy(
                h_sp.at[src_all.at[pl.ds(g * CHUNK, HALF)]],
                rows[b].at[pl.ds(0, HALF)], g_sem[b])
            pltpu.async_copy(
                h_sp.at[src_all.at[pl.ds(g * CHUNK + HALF, HALF)]],
                rows[b].at[pl.ds(HALF, HALF)], g2_sem[b])

        def gather_wait(g, b):
            pltpu.make_async_copy(
                h_sp.at[src_all.at[pl.ds(g * CHUNK, HALF)]],
                rows[b].at[pl.ds(0, HALF)], g_sem[b]).wait()
            pltpu.make_async_copy(
                h_sp.at[src_all.at[pl.ds(g * CHUNK + HALF, HALF)]],
                rows[b].at[pl.ds(HALF, HALF)], g2_sem[b]).wait()

        def scatter_start(g, b):
            di = dst_all.at[pl.ds(g * CHUNK, CHUNK)]
            pltpu.async_copy(rows[b], acc.at[di], s_sem[b], add=True)
            pltpu.async_copy(ones_v, cnt.at[di], c_sem[b], add=True)

        def scatter_wait(g, b):
            di = dst_all.at[pl.ds(g * CHUNK, CHUNK)]
            pltpu.make_async_copy(rows[b], acc.at[di], s_sem[b]).wait()
            pltpu.make_async_copy(ones_v, cnt.at[di], c_sem[b]).wait()

        gather_start(0, 0)
        gather_wait(0, 0)
        scatter_start(0, 0)
        gather_start(1, 1)

        def pair_body(p, _):
            g = 2 * p + 1
            gather_wait(g, 1)
            scatter_start(g, 1)
            scatter_wait(g - 1, 0)
            gather_start(g + 1, 0)
            gather_wait(g + 1, 0)
            scatter_start(g + 1, 0)
            scatter_wait(g, 1)
            gather_start(g + 2, 1)
            return 0
        lax.fori_loop(0, (N_CHUNKS - 2) // 2, pair_body, 0)

        gather_wait(N_CHUNKS - 1, 1)
        scatter_start(N_CHUNKS - 1, 1)
        scatter_wait(N_CHUNKS - 2, 0)
        scatter_wait(N_CHUNKS - 1, 1)
        plsc.subcore_barrier()

        @pl.when(sid < NS - 1)
        def _():
            pltpu.sync_copy(acc.at[pl.ds(sid * ROWS_PER_TILE, ROWS_PER_TILE)],
                            out_hbm.at[pl.ds(sid * ROWS_PER_TILE, ROWS_PER_TILE)])
            pltpu.sync_copy(cnt.at[pl.ds(sid * ROWS_PER_TILE, ROWS_PER_TILE)],
                            cnt_hbm.at[pl.ds(sid * ROWS_PER_TILE, ROWS_PER_TILE)])

        @pl.when(sid == NS - 1)
        def _():
            pltpu.sync_copy(acc.at[pl.ds((NS - 1) * ROWS_PER_TILE, LAST_ROWS)],
                            out_hbm.at[pl.ds((NS - 1) * ROWS_PER_TILE, LAST_ROWS)])
            pltpu.sync_copy(cnt.at[pl.ds((NS - 1) * ROWS_PER_TILE, LAST_ROWS)],
                            cnt_hbm.at[pl.ds((NS - 1) * ROWS_PER_TILE, LAST_ROWS)])

    @pl.when(cid == 0)
    def _():
        process(h1_hbm, src1_hbm, dst1_hbm, out1_hbm, cnt1_hbm)

    @pl.when(cid == 1)
    def _():
        process(h2_hbm, src2_hbm, dst2_hbm, out2_hbm, cnt2_hbm)


def kernel(x_account, x_transaction, edge_index_initiates, edge_index_receives,
           W_l1, b_l1, W_r1, W_l2, b_l2, W_r2):
    n_pad = E_PAD - N_EDGES
    src1 = jnp.concatenate([edge_index_initiates[0].astype(jnp.int32),
                            jnp.zeros((n_pad,), jnp.int32)])
    dst1 = jnp.concatenate([edge_index_initiates[1].astype(jnp.int32),
                            jnp.full((n_pad,), N_NODE, jnp.int32)])
    src2 = jnp.concatenate([edge_index_receives[0].astype(jnp.int32),
                            jnp.zeros((n_pad,), jnp.int32)])
    dst2 = jnp.concatenate([edge_index_receives[1].astype(jnp.int32),
                            jnp.full((n_pad,), N_NODE, jnp.int32)])

    stage1 = pl.pallas_call(
        _stage1_body,
        grid=(N_NODE // BLK,),
        in_specs=[
            pl.BlockSpec((BLK, D_FEAT), lambda i: (i, 0)),
            pl.BlockSpec((BLK, D_FEAT), lambda i: (i, 0)),
            pl.BlockSpec((D_FEAT, D_OUT), lambda i: (0, 0)),
            pl.BlockSpec((1, D_OUT), lambda i: (0, 0)),
            pl.BlockSpec((D_FEAT, D_OUT), lambda i: (0, 0)),
            pl.BlockSpec((D_FEAT, D_OUT), lambda i: (0, 0)),
            pl.BlockSpec((1, D_OUT), lambda i: (0, 0)),
            pl.BlockSpec((D_FEAT, D_OUT), lambda i: (0, 0)),
        ],
        out_specs=[
            pl.BlockSpec((BLK, D_OUT), lambda i: (i, 0)),
            pl.BlockSpec((BLK, D_OUT), lambda i: (i, 0)),
            pl.BlockSpec((BLK, D_OUT), lambda i: (i, 0)),
            pl.BlockSpec((BLK, D_OUT), lambda i: (i, 0)),
        ],
        out_shape=[
            jax.ShapeDtypeStruct((N_NODE, D_OUT), jnp.float32),
            jax.ShapeDtypeStruct((N_NODE, D_OUT), jnp.float32),
            jax.ShapeDtypeStruct((N_NODE, D_OUT), jnp.float32),
            jax.ShapeDtypeStruct((N_NODE, D_OUT), jnp.float32),
        ],
    )
    h1, h2, r1, r2 = stage1(x_account, x_transaction,
                            W_l1, b_l1.reshape(1, D_OUT), W_r1,
                            W_l2, b_l2.reshape(1, D_OUT), W_r2)

    seg = pl.kernel(
        _segment_kernel,
        out_type=[
            jax.ShapeDtypeStruct((N_NODE, D_OUT), jnp.float32),
            jax.ShapeDtypeStruct((N_NODE,), jnp.float32),
            jax.ShapeDtypeStruct((N_NODE, D_OUT), jnp.float32),
            jax.ShapeDtypeStruct((N_NODE,), jnp.float32),
        ],
        mesh=plsc.VectorSubcoreMesh(core_axis_name="c", subcore_axis_name="s",
                                    num_cores=NC, num_subcores=NS),
        compiler_params=pltpu.CompilerParams(use_tc_tiling_on_sc=False),
        scratch_types=[
            pltpu.VMEM_SHARED((ACC_ROWS, D_OUT), jnp.float32),
            pltpu.VMEM_SHARED((ACC_ROWS,), jnp.float32),
            pltpu.VMEM_SHARED((N_NODE, D_OUT), jnp.float32),
            pltpu.VMEM((EDGES_PER_TILE,), jnp.int32),
            pltpu.VMEM((EDGES_PER_TILE,), jnp.int32),
            pltpu.VMEM((CHUNK, D_OUT), jnp.float32),
            pltpu.VMEM((CHUNK, D_OUT), jnp.float32),
            pltpu.VMEM((CHUNK,), jnp.float32),
            pltpu.VMEM((ZROWS,), jnp.float32),
            pltpu.SemaphoreType.DMA,
            pltpu.SemaphoreType.DMA,
            pltpu.SemaphoreType.DMA,
            pltpu.SemaphoreType.DMA,
            pltpu.SemaphoreType.DMA,
            pltpu.SemaphoreType.DMA,
            pltpu.SemaphoreType.DMA,
            pltpu.SemaphoreType.DMA,
        ],
    )
    acc1, cnt1, acc2, cnt2 = seg(h1, src1, dst1, h2, src2, dst2)

    stage3 = pl.pallas_call(
        _stage3_body,
        grid=(N_NODE // BLK,),
        in_specs=[
            pl.BlockSpec((BLK, D_OUT), lambda i: (i, 0)),
            pl.BlockSpec((BLK, 1), lambda i: (i, 0)),
            pl.BlockSpec((BLK, D_OUT), lambda i: (i, 0)),
            pl.BlockSpec((BLK, D_OUT), lambda i: (i, 0)),
            pl.BlockSpec((BLK, 1), lambda i: (i, 0)),
            pl.BlockSpec((BLK, D_OUT), lambda i: (i, 0)),
        ],
        out_specs=[
            pl.BlockSpec((BLK, D_OUT), lambda i: (i, 0)),
            pl.BlockSpec((BLK, D_OUT), lambda i: (i, 0)),
        ],
        out_shape=[
            jax.ShapeDtypeStruct((N_NODE, D_OUT), jnp.float32),
            jax.ShapeDtypeStruct((N_NODE, D_OUT), jnp.float32),
        ],
    )
    out_transaction, out_account = stage3(acc1, cnt1.reshape(N_NODE, 1), r1,
                                          acc2, cnt2.reshape(N_NODE, 1), r2)
    return (out_transaction, out_account)


# speedup vs baseline: 1.0022x; 1.0022x over previous
import functools

import jax
import jax.numpy as jnp
from jax import lax
from jax.experimental import pallas as pl
from jax.experimental.pallas import tpu as pltpu
from jax.experimental.pallas import tpu_sc as plsc

N_NODE = 10000
D_FEAT = 128
D_OUT = 16
N_EDGES = 320000

NC, NS = 2, 16
CHUNK = 2048
EDGES_PER_TILE = 20480
E_PAD = EDGES_PER_TILE * NS
N_CHUNKS = EDGES_PER_TILE // CHUNK
ROWS_PER_TILE = 624
LAST_ROWS = N_NODE - ROWS_PER_TILE * (NS - 1)
ACC_ROWS = 10240
ZROWS = ACC_ROWS // NS
BLK = 2000


def _stage1_body(xa_ref, xt_ref, wl1_ref, bl1_ref, wr1_ref, wl2_ref, bl2_ref,
                 wr2_ref, h1_ref, h2_ref, r1_ref, r2_ref):
    xa = xa_ref[...]
    xt = xt_ref[...]
    h1_ref[...] = jnp.dot(xa, wl1_ref[...], preferred_element_type=jnp.float32)
    h2_ref[...] = jnp.dot(xt, wl2_ref[...], preferred_element_type=jnp.float32)
    r1_ref[...] = jnp.dot(xt, wr1_ref[...],
                          preferred_element_type=jnp.float32) + bl1_ref[...]
    r2_ref[...] = jnp.dot(xa, wr2_ref[...],
                          preferred_element_type=jnp.float32) + bl2_ref[...]


def _stage3_body(acc1_ref, cnt1_ref, r1_ref, acc2_ref, cnt2_ref, r2_ref,
                 out_t_ref, out_a_ref):
    out_t_ref[...] = (acc1_ref[...] / jnp.maximum(cnt1_ref[...], 1.0)
                      + r1_ref[...])
    out_a_ref[...] = (acc2_ref[...] / jnp.maximum(cnt2_ref[...], 1.0)
                      + r2_ref[...])


def _segment_kernel(h1_hbm, src1_hbm, dst1_hbm, h2_hbm, src2_hbm, dst2_hbm,
                    out1_hbm, cnt1_hbm, out2_hbm, cnt2_hbm,
                    acc, cnt, h_sp, src_all, dst_all, rows_v0, rows_v1, ones_v,
                    zero_c, g_sem0, g_sem1, s_sem0, s_sem1,
                    c_sem0, c_sem1):
    cid = lax.axis_index("c")
    sid = lax.axis_index("s")

    z16 = jnp.zeros((16,), jnp.float32)
    o16 = jnp.ones((16,), jnp.float32)

    def zbody(i, _):
        rows_v0[i, pl.ds(0, 16)] = z16
        return 0
    lax.fori_loop(0, ZROWS, zbody, 0)

    def zcbody(i, _):
        zero_c[pl.ds(i * 16, 16)] = z16
        return 0
    lax.fori_loop(0, ZROWS // 16, zcbody, 0)

    def obody(i, _):
        ones_v[pl.ds(i * 16, 16)] = o16
        return 0
    lax.fori_loop(0, CHUNK // 16, obody, 0)

    pltpu.sync_copy(rows_v0.at[pl.ds(0, ZROWS)], acc.at[pl.ds(sid * ZROWS, ZROWS)])
    pltpu.sync_copy(zero_c, cnt.at[pl.ds(sid * ZROWS, ZROWS)])
    plsc.subcore_barrier()

    def process(h_hbm, src_hbm, dst_hbm, out_hbm, cnt_hbm):
        base = sid * EDGES_PER_TILE
        pltpu.sync_copy(src_hbm.at[pl.ds(base, EDGES_PER_TILE)], src_all)
        pltpu.sync_copy(dst_hbm.at[pl.ds(base, EDGES_PER_TILE)], dst_all)

        @pl.when(sid < NS - 1)
        def _():
            pltpu.sync_copy(h_hbm.at[pl.ds(sid * ROWS_PER_TILE, ROWS_PER_TILE)],
                            h_sp.at[pl.ds(sid * ROWS_PER_TILE, ROWS_PER_TILE)])

        @pl.when(sid == NS - 1)
        def _():
            pltpu.sync_copy(h_hbm.at[pl.ds((NS - 1) * ROWS_PER_TILE, LAST_ROWS)],
                            h_sp.at[pl.ds((NS - 1) * ROWS_PER_TILE, LAST_ROWS)])
        plsc.subcore_barrier()

        rows = (rows_v0, rows_v1)
        g_sem = (g_sem0, g_sem1)
        s_sem = (s_sem0, s_sem1)
        c_sem = (c_sem0, c_sem1)

        def gather_start(g, b):
            pltpu.async_copy(h_sp.at[src_all.at[pl.ds(g * CHUNK, CHUNK)]],
                             rows[b], g_sem[b])

        def gather_wait(g, b):
            pltpu.make_async_copy(
                h_sp.at[src_all.at[pl.ds(g * CHUNK, CHUNK)]],
                rows[b], g_sem[b]).wait()

        def scatter_start(g, b):
            di = dst_all.at[pl.ds(g * CHUNK, CHUNK)]
            pltpu.async_copy(rows[b], acc.at[di], s_sem[b], add=True)
            pltpu.async_copy(ones_v, cnt.at[di], c_sem[b], add=True)

        def scatter_wait(g, b):
            di = dst_all.at[pl.ds(g * CHUNK, CHUNK)]
            pltpu.make_async_copy(rows[b], acc.at[di], s_sem[b]).wait()
            pltpu.make_async_copy(ones_v, cnt.at[di], c_sem[b]).wait()

        gather_start(0, 0)
        gather_wait(0, 0)
        scatter_start(0, 0)
        gather_start(1, 1)

        def pair_body(p, _):
            g = 2 * p + 1
            gather_wait(g, 1)
            scatter_start(g, 1)
            scatter_wait(g - 1, 0)
            gather_start(g + 1, 0)
            gather_wait(g + 1, 0)
            scatter_start(g + 1, 0)
            scatter_wait(g, 1)
            gather_start(g + 2, 1)
            return 0
        lax.fori_loop(0, (N_CHUNKS - 2) // 2, pair_body, 0)

        gather_wait(N_CHUNKS - 1, 1)
        scatter_start(N_CHUNKS - 1, 1)
        scatter_wait(N_CHUNKS - 2, 0)
        scatter_wait(N_CHUNKS - 1, 1)
        plsc.subcore_barrier()

        @pl.when(sid < NS - 1)
        def _():
            pltpu.sync_copy(acc.at[pl.ds(sid * ROWS_PER_TILE, ROWS_PER_TILE)],
                            out_hbm.at[pl.ds(sid * ROWS_PER_TILE, ROWS_PER_TILE)])
            pltpu.sync_copy(cnt.at[pl.ds(sid * ROWS_PER_TILE, ROWS_PER_TILE)],
                            cnt_hbm.at[pl.ds(sid * ROWS_PER_TILE, ROWS_PER_TILE)])

        @pl.when(sid == NS - 1)
        def _():
            pltpu.sync_copy(acc.at[pl.ds((NS - 1) * ROWS_PER_TILE, LAST_ROWS)],
                            out_hbm.at[pl.ds((NS - 1) * ROWS_PER_TILE, LAST_ROWS)])
            pltpu.sync_copy(cnt.at[pl.ds((NS - 1) * ROWS_PER_TILE, LAST_ROWS)],
                            cnt_hbm.at[pl.ds((NS - 1) * ROWS_PER_TILE, LAST_ROWS)])

    @pl.when(cid == 0)
    def _():
        process(h1_hbm, src1_hbm, dst1_hbm, out1_hbm, cnt1_hbm)

    @pl.when(cid == 1)
    def _():
        process(h2_hbm, src2_hbm, dst2_hbm, out2_hbm, cnt2_hbm)


def kernel(x_account, x_transaction, edge_index_initiates, edge_index_receives,
           W_l1, b_l1, W_r1, W_l2, b_l2, W_r2):
    n_pad = E_PAD - N_EDGES
    src1 = jnp.concatenate([edge_index_initiates[0].astype(jnp.int32),
                            jnp.zeros((n_pad,), jnp.int32)])
    dst1 = jnp.concatenate([edge_index_initiates[1].astype(jnp.int32),
                            jnp.full((n_pad,), N_NODE, jnp.int32)])
    src2 = jnp.concatenate([edge_index_receives[0].astype(jnp.int32),
                            jnp.zeros((n_pad,), jnp.int32)])
    dst2 = jnp.concatenate([edge_index_receives[1].astype(jnp.int32),
                            jnp.full((n_pad,), N_NODE, jnp.int32)])

    stage1 = pl.pallas_call(
        _stage1_body,
        grid=(N_NODE // BLK,),
        in_specs=[
            pl.BlockSpec((BLK, D_FEAT), lambda i: (i, 0)),
            pl.BlockSpec((BLK, D_FEAT), lambda i: (i, 0)),
            pl.BlockSpec((D_FEAT, D_OUT), lambda i: (0, 0)),
            pl.BlockSpec((1, D_OUT), lambda i: (0, 0)),
            pl.BlockSpec((D_FEAT, D_OUT), lambda i: (0, 0)),
            pl.BlockSpec((D_FEAT, D_OUT), lambda i: (0, 0)),
            pl.BlockSpec((1, D_OUT), lambda i: (0, 0)),
            pl.BlockSpec((D_FEAT, D_OUT), lambda i: (0, 0)),
        ],
        out_specs=[
            pl.BlockSpec((BLK, D_OUT), lambda i: (i, 0)),
            pl.BlockSpec((BLK, D_OUT), lambda i: (i, 0)),
            pl.BlockSpec((BLK, D_OUT), lambda i: (i, 0)),
            pl.BlockSpec((BLK, D_OUT), lambda i: (i, 0)),
        ],
        out_shape=[
            jax.ShapeDtypeStruct((N_NODE, D_OUT), jnp.float32),
            jax.ShapeDtypeStruct((N_NODE, D_OUT), jnp.float32),
            jax.ShapeDtypeStruct((N_NODE, D_OUT), jnp.float32),
            jax.ShapeDtypeStruct((N_NODE, D_OUT), jnp.float32),
        ],
    )
    h1, h2, r1, r2 = stage1(x_account, x_transaction,
                            W_l1, b_l1.reshape(1, D_OUT), W_r1,
                            W_l2, b_l2.reshape(1, D_OUT), W_r2)

    seg = pl.kernel(
        _segment_kernel,
        out_type=[
            jax.ShapeDtypeStruct((N_NODE, D_OUT), jnp.float32),
            jax.ShapeDtypeStruct((N_NODE,), jnp.float32),
            jax.ShapeDtypeStruct((N_NODE, D_OUT), jnp.float32),
            jax.ShapeDtypeStruct((N_NODE,), jnp.float32),
        ],
        mesh=plsc.VectorSubcoreMesh(core_axis_name="c", subcore_axis_name="s",
                                    num_cores=NC, num_subcores=NS),
        compiler_params=pltpu.CompilerParams(use_tc_tiling_on_sc=False),
        scratch_types=[
            pltpu.VMEM_SHARED((ACC_ROWS, D_OUT), jnp.float32),
            pltpu.VMEM_SHARED((ACC_ROWS,), jnp.float32),
            pltpu.VMEM_SHARED((N_NODE, D_OUT), jnp.float32),
            pltpu.VMEM((EDGES_PER_TILE,), jnp.int32),
            pltpu.VMEM((EDGES_PER_TILE,), jnp.int32),
            pltpu.VMEM((CHUNK, D_OUT), jnp.float32),
            pltpu.VMEM((CHUNK, D_OUT), jnp.float32),
            pltpu.VMEM((CHUNK,), jnp.float32),
            pltpu.VMEM((ZROWS,), jnp.float32),
            pltpu.SemaphoreType.DMA,
            pltpu.SemaphoreType.DMA,
            pltpu.SemaphoreType.DMA,
            pltpu.SemaphoreType.DMA,
            pltpu.SemaphoreType.DMA,
            pltpu.SemaphoreType.DMA,
        ],
    )
    acc1, cnt1, acc2, cnt2 = seg(h1, src1, dst1, h2, src2, dst2)

    stage3 = pl.pallas_call(
        _stage3_body,
        grid=(N_NODE // BLK,),
        in_specs=[
            pl.BlockSpec((BLK, D_OUT), lambda i: (i, 0)),
            pl.BlockSpec((BLK, 1), lambda i: (i, 0)),
            pl.BlockSpec((BLK, D_OUT), lambda i: (i, 0)),
            pl.BlockSpec((BLK, D_OUT), lambda i: (i, 0)),
            pl.BlockSpec((BLK, 1), lambda i: (i, 0)),
            pl.BlockSpec((BLK, D_OUT), lambda i: (i, 0)),
        ],
        out_specs=[
            pl.BlockSpec((BLK, D_OUT), lambda i: (i, 0)),
            pl.BlockSpec((BLK, D_OUT), lambda i: (i, 0)),
        ],
        out_shape=[
            jax.ShapeDtypeStruct((N_NODE, D_OUT), jnp.float32),
            jax.ShapeDtypeStruct((N_NODE, D_OUT), jnp.float32),
        ],
    )
    out_transaction, out_account = stage3(acc1, cnt1.reshape(N_NODE, 1), r1,
                                          acc2, cnt2.reshape(N_NODE, 1), r2)
    return (out_transaction, out_account)


# speedup vs baseline: 1.3997x; 1.3967x over previous
import jax
import jax.numpy as jnp
from jax import lax
from jax.experimental import pallas as pl
from jax.experimental.pallas import tpu as pltpu
from jax.experimental.pallas import tpu_sc as plsc

N_NODE = 10000
D_FEAT = 128
D_OUT = 16
N_EDGES = 320000

NC, NS = 2, 16
CHUNK = 2000
EDGES_PER_TILE = N_EDGES // NS
N_CHUNKS = EDGES_PER_TILE // CHUNK
ROWS_PER_TILE = 624
LAST_ROWS = N_NODE - ROWS_PER_TILE * (NS - 1)
BLK = 2000


def _stage1_body(xa_ref, xt_ref, wl1_ref, bl1_ref, wr1_ref, wl2_ref, bl2_ref,
                 wr2_ref, h1_ref, h2_ref, r1_ref, r2_ref):
    xa = xa_ref[...]
    xt = xt_ref[...]
    h1_ref[...] = jnp.dot(xa, wl1_ref[...], preferred_element_type=jnp.float32)
    h2_ref[...] = jnp.dot(xt, wl2_ref[...], preferred_element_type=jnp.float32)
    r1_ref[...] = jnp.dot(xt, wr1_ref[...],
                          preferred_element_type=jnp.float32) + bl1_ref[...]
    r2_ref[...] = jnp.dot(xa, wr2_ref[...],
                          preferred_element_type=jnp.float32) + bl2_ref[...]


def _segment_kernel(h1_hbm, e1_hbm, h2_hbm, e2_hbm, r1_hbm, r2_hbm,
                    out1_hbm, out2_hbm,
                    acc, cnt, h_sp, src_all, dst_all, rows_v0, rows_v1, ones_v,
                    cnt_v, g_sem0, g_sem1, s_sem0, s_sem1, c_sem0, c_sem1):
    cid = lax.axis_index("c")
    sid = lax.axis_index("s")

    z16 = jnp.zeros((16,), jnp.float32)
    o16 = jnp.ones((16,), jnp.float32)

    def zbody(i, _):
        rows_v0[i, pl.ds(0, 16)] = z16
        return 0
    lax.fori_loop(0, LAST_ROWS, zbody, 0)

    def zcbody(i, _):
        cnt_v[pl.ds(i * 16, 16)] = z16
        return 0
    lax.fori_loop(0, LAST_ROWS // 16, zcbody, 0)

    def obody(i, _):
        ones_v[pl.ds(i * 16, 16)] = o16
        return 0
    lax.fori_loop(0, CHUNK // 16, obody, 0)

    def zero_acc(rbase, rws):
        pltpu.sync_copy(rows_v0.at[pl.ds(0, rws)], acc.at[pl.ds(rbase, rws)])
        pltpu.sync_copy(cnt_v.at[pl.ds(0, rws)], cnt.at[pl.ds(rbase, rws)])

    @pl.when(sid < NS - 1)
    def _():
        zero_acc(sid * ROWS_PER_TILE, ROWS_PER_TILE)

    @pl.when(sid == NS - 1)
    def _():
        zero_acc((NS - 1) * ROWS_PER_TILE, LAST_ROWS)

    def process(h_hbm, e_hbm, r_hbm, out_hbm):
        base = sid * EDGES_PER_TILE
        pltpu.sync_copy(e_hbm.at[0, pl.ds(base, EDGES_PER_TILE)], src_all)
        pltpu.sync_copy(e_hbm.at[1, pl.ds(base, EDGES_PER_TILE)], dst_all)

        @pl.when(sid < NS - 1)
        def _():
            pltpu.sync_copy(h_hbm.at[pl.ds(sid * ROWS_PER_TILE, ROWS_PER_TILE)],
                            h_sp.at[pl.ds(sid * ROWS_PER_TILE, ROWS_PER_TILE)])

        @pl.when(sid == NS - 1)
        def _():
            pltpu.sync_copy(h_hbm.at[pl.ds((NS - 1) * ROWS_PER_TILE, LAST_ROWS)],
                            h_sp.at[pl.ds((NS - 1) * ROWS_PER_TILE, LAST_ROWS)])
        plsc.subcore_barrier()

        rows = (rows_v0, rows_v1)
        g_sem = (g_sem0, g_sem1)
        s_sem = (s_sem0, s_sem1)
        c_sem = (c_sem0, c_sem1)

        def gather_start(g, b):
            pltpu.async_copy(h_sp.at[src_all.at[pl.ds(g * CHUNK, CHUNK)]],
                             rows[b], g_sem[b])

        def gather_wait(g, b):
            pltpu.make_async_copy(
                h_sp.at[src_all.at[pl.ds(g * CHUNK, CHUNK)]],
                rows[b], g_sem[b]).wait()

        def scatter_start(g, b):
            di = dst_all.at[pl.ds(g * CHUNK, CHUNK)]
            pltpu.async_copy(rows[b], acc.at[di], s_sem[b], add=True)
            pltpu.async_copy(ones_v, cnt.at[di], c_sem[b], add=True)

        def scatter_wait(g, b):
            di = dst_all.at[pl.ds(g * CHUNK, CHUNK)]
            pltpu.make_async_copy(rows[b], acc.at[di], s_sem[b]).wait()
            pltpu.make_async_copy(ones_v, cnt.at[di], c_sem[b]).wait()

        gather_start(0, 0)
        gather_wait(0, 0)
        scatter_start(0, 0)
        gather_start(1, 1)

        def pair_body(p, _):
            g = 2 * p + 1
            gather_wait(g, 1)
            scatter_start(g, 1)
            scatter_wait(g - 1, 0)
            gather_start(g + 1, 0)
            gather_wait(g + 1, 0)
            scatter_start(g + 1, 0)
            scatter_wait(g, 1)
            gather_start(g + 2, 1)
            return 0
        lax.fori_loop(0, (N_CHUNKS - 2) // 2, pair_body, 0)

        gather_wait(N_CHUNKS - 1, 1)
        scatter_start(N_CHUNKS - 1, 1)
        scatter_wait(N_CHUNKS - 2, 0)
        scatter_wait(N_CHUNKS - 1, 1)
        plsc.subcore_barrier()

        def finish(rbase, rws):
            pltpu.sync_copy(acc.at[pl.ds(rbase, rws)],
                            rows_v0.at[pl.ds(0, rws)])
            pltpu.sync_copy(cnt.at[pl.ds(rbase, rws)],
                            cnt_v.at[pl.ds(0, rws)])
            pltpu.sync_copy(r_hbm.at[pl.ds(rbase, rws)],
                            rows_v1.at[pl.ds(0, rws)])

            def grp(j, _):
                c16 = cnt_v[pl.ds(j * 16, 16)]
                ones_v[pl.ds(j * 16, 16)] = 1.0 / jnp.maximum(c16, 1.0)
                return 0
            lax.fori_loop(0, rws // 16, grp, 0)

            def rowb(i, _):
                s = ones_v[pl.ds(i, 16)][0]
                rows_v0[i, pl.ds(0, 16)] = (rows_v0[i, pl.ds(0, 16)] * s
                                            + rows_v1[i, pl.ds(0, 16)])
                return 0
            lax.fori_loop(0, rws, rowb, 0)
            pltpu.sync_copy(rows_v0.at[pl.ds(0, rws)],
                            out_hbm.at[pl.ds(rbase, rws)])

        @pl.when(sid < NS - 1)
        def _():
            finish(sid * ROWS_PER_TILE, ROWS_PER_TILE)

        @pl.when(sid == NS - 1)
        def _():
            finish((NS - 1) * ROWS_PER_TILE, LAST_ROWS)

    @pl.when(cid == 0)
    def _():
        process(h1_hbm, e1_hbm, r1_hbm, out1_hbm)

    @pl.when(cid == 1)
    def _():
        process(h2_hbm, e2_hbm, r2_hbm, out2_hbm)


def kernel(x_account, x_transaction, edge_index_initiates, edge_index_receives,
           W_l1, b_l1, W_r1, W_l2, b_l2, W_r2):
    e1 = edge_index_initiates.astype(jnp.int32)
    e2 = edge_index_receives.astype(jnp.int32)

    stage1 = pl.pallas_call(
        _stage1_body,
        grid=(N_NODE // BLK,),
        in_specs=[
            pl.BlockSpec((BLK, D_FEAT), lambda i: (i, 0)),
            pl.BlockSpec((BLK, D_FEAT), lambda i: (i, 0)),
            pl.BlockSpec((D_FEAT, D_OUT), lambda i: (0, 0)),
            pl.BlockSpec((1, D_OUT), lambda i: (0, 0)),
            pl.BlockSpec((D_FEAT, D_OUT), lambda i: (0, 0)),
            pl.BlockSpec((D_FEAT, D_OUT), lambda i: (0, 0)),
            pl.BlockSpec((1, D_OUT), lambda i: (0, 0)),
            pl.BlockSpec((D_FEAT, D_OUT), lambda i: (0, 0)),
        ],
        out_specs=[
            pl.BlockSpec((BLK, D_OUT), lambda i: (i, 0)),
            pl.BlockSpec((BLK, D_OUT), lambda i: (i, 0)),
            pl.BlockSpec((BLK, D_OUT), lambda i: (i, 0)),
            pl.BlockSpec((BLK, D_OUT), lambda i: (i, 0)),
        ],
        out_shape=[
            jax.ShapeDtypeStruct((N_NODE, D_OUT), jnp.float32),
            jax.ShapeDtypeStruct((N_NODE, D_OUT), jnp.float32),
            jax.ShapeDtypeStruct((N_NODE, D_OUT), jnp.float32),
            jax.ShapeDtypeStruct((N_NODE, D_OUT), jnp.float32),
        ],
    )
    h1, h2, r1, r2 = stage1(x_account, x_transaction,
                            W_l1, b_l1.reshape(1, D_OUT), W_r1,
                            W_l2, b_l2.reshape(1, D_OUT), W_r2)

    seg = pl.kernel(
        _segment_kernel,
        out_type=[
            jax.ShapeDtypeStruct((N_NODE, D_OUT), jnp.float32),
            jax.ShapeDtypeStruct((N_NODE, D_OUT), jnp.float32),
        ],
        mesh=plsc.VectorSubcoreMesh(core_axis_name="c", subcore_axis_name="s",
                                    num_cores=NC, num_subcores=NS),
        compiler_params=pltpu.CompilerParams(use_tc_tiling_on_sc=False),
        scratch_types=[
            pltpu.VMEM_SHARED((N_NODE, D_OUT), jnp.float32),
            pltpu.VMEM_SHARED((N_NODE,), jnp.float32),
            pltpu.VMEM_SHARED((N_NODE, D_OUT), jnp.float32),
            pltpu.VMEM((EDGES_PER_TILE,), jnp.int32),
            pltpu.VMEM((EDGES_PER_TILE,), jnp.int32),
            pltpu.VMEM((CHUNK, D_OUT), jnp.float32),
            pltpu.VMEM((CHUNK, D_OUT), jnp.float32),
            pltpu.VMEM((CHUNK,), jnp.float32),
            pltpu.VMEM((LAST_ROWS,), jnp.float32),
            pltpu.SemaphoreType.DMA,
            pltpu.SemaphoreType.DMA,
            pltpu.SemaphoreType.DMA,
            pltpu.SemaphoreType.DMA,
            pltpu.SemaphoreType.DMA,
            pltpu.SemaphoreType.DMA,
        ],
    )
    out_transaction, out_account = seg(h1, e1, h2, e2, r1, r2)
    return (out_transaction, out_account)


# speedup vs baseline: 1.4478x; 1.0344x over previous
import jax
import jax.numpy as jnp
from jax import lax
from jax.experimental import pallas as pl
from jax.experimental.pallas import tpu as pltpu
from jax.experimental.pallas import tpu_sc as plsc

N_NODE = 10000
D_FEAT = 128
D_OUT = 16
N_EDGES = 320000

NC, NS = 2, 16
CHUNK = 2000
EDGES_PER_TILE = N_EDGES // NS
N_CHUNKS = EDGES_PER_TILE // CHUNK
ROWS_PER_TILE = 624
LAST_ROWS = N_NODE - ROWS_PER_TILE * (NS - 1)
BLK = 2000


def _stage1_body(xa_ref, xt_ref, wl1_ref, bl1_ref, wr1_ref, wl2_ref, bl2_ref,
                 wr2_ref, h1_ref, h2_ref, r1_ref, r2_ref):
    xa = xa_ref[...]
    xt = xt_ref[...]
    h1_ref[...] = jnp.dot(xa, wl1_ref[...], preferred_element_type=jnp.float32)
    h2_ref[...] = jnp.dot(xt, wl2_ref[...], preferred_element_type=jnp.float32)
    r1_ref[...] = jnp.dot(xt, wr1_ref[...],
                          preferred_element_type=jnp.float32) + bl1_ref[...]
    r2_ref[...] = jnp.dot(xa, wr2_ref[...],
                          preferred_element_type=jnp.float32) + bl2_ref[...]


def _segment_kernel(h1_hbm, e1_hbm, h2_hbm, e2_hbm, r1_hbm, r2_hbm,
                    out1_hbm, out2_hbm,
                    acc, cnt, h_sp, src_all, dst_all, rows_v0, rows_v1, ones_v,
                    cnt_v, g_sem0, g_sem1, s_sem0, s_sem1, c_sem0, c_sem1):
    cid = lax.axis_index("c")
    sid = lax.axis_index("s")

    def zero_init():
        z16 = jnp.zeros((16,), jnp.float32)
        o16 = jnp.ones((16,), jnp.float32)

        def zbody(i, _):
            rows_v0[i, pl.ds(0, 16)] = z16
            return 0
        lax.fori_loop(0, LAST_ROWS, zbody, 0)

        def zcbody(i, _):
            cnt_v[pl.ds(i * 16, 16)] = z16
            return 0
        lax.fori_loop(0, LAST_ROWS // 16, zcbody, 0)

        def obody(i, _):
            ones_v[pl.ds(i * 16, 16)] = o16
            return 0
        lax.fori_loop(0, CHUNK // 16, obody, 0)

        def zero_acc(rbase, rws):
            pltpu.sync_copy(rows_v0.at[pl.ds(0, rws)], acc.at[pl.ds(rbase, rws)])
            pltpu.sync_copy(cnt_v.at[pl.ds(0, rws)], cnt.at[pl.ds(rbase, rws)])

        @pl.when(sid < NS - 1)
        def _():
            zero_acc(sid * ROWS_PER_TILE, ROWS_PER_TILE)

        @pl.when(sid == NS - 1)
        def _():
            zero_acc((NS - 1) * ROWS_PER_TILE, LAST_ROWS)

    def process(h_hbm, e_hbm, r_hbm, out_hbm):
        base = sid * EDGES_PER_TILE
        pltpu.async_copy(e_hbm.at[0, pl.ds(base, EDGES_PER_TILE)], src_all,
                         s_sem0)
        pltpu.async_copy(e_hbm.at[1, pl.ds(base, EDGES_PER_TILE)], dst_all,
                         s_sem1)

        @pl.when(sid < NS - 1)
        def _():
            pltpu.async_copy(h_hbm.at[pl.ds(sid * ROWS_PER_TILE, ROWS_PER_TILE)],
                             h_sp.at[pl.ds(sid * ROWS_PER_TILE, ROWS_PER_TILE)],
                             c_sem0)

        @pl.when(sid == NS - 1)
        def _():
            pltpu.async_copy(h_hbm.at[pl.ds((NS - 1) * ROWS_PER_TILE, LAST_ROWS)],
                             h_sp.at[pl.ds((NS - 1) * ROWS_PER_TILE, LAST_ROWS)],
                             c_sem0)

        zero_init()
        pltpu.make_async_copy(e_hbm.at[0, pl.ds(base, EDGES_PER_TILE)],
                              src_all, s_sem0).wait()
        pltpu.make_async_copy(e_hbm.at[1, pl.ds(base, EDGES_PER_TILE)],
                              dst_all, s_sem1).wait()

        @pl.when(sid < NS - 1)
        def _():
            pltpu.make_async_copy(
                h_hbm.at[pl.ds(sid * ROWS_PER_TILE, ROWS_PER_TILE)],
                h_sp.at[pl.ds(sid * ROWS_PER_TILE, ROWS_PER_TILE)],
                c_sem0).wait()

        @pl.when(sid == NS - 1)
        def _():
            pltpu.make_async_copy(
                h_hbm.at[pl.ds((NS - 1) * ROWS_PER_TILE, LAST_ROWS)],
                h_sp.at[pl.ds((NS - 1) * ROWS_PER_TILE, LAST_ROWS)],
                c_sem0).wait()
        plsc.subcore_barrier()

        rows = (rows_v0, rows_v1)
        g_sem = (g_sem0, g_sem1)
        s_sem = (s_sem0, s_sem1)
        c_sem = (c_sem0, c_sem1)

        def gather_start(g, b):
            pltpu.async_copy(h_sp.at[src_all.at[pl.ds(g * CHUNK, CHUNK)]],
                             rows[b], g_sem[b])

        def gather_wait(g, b):
            pltpu.make_async_copy(
                h_sp.at[src_all.at[pl.ds(g * CHUNK, CHUNK)]],
                rows[b], g_sem[b]).wait()

        def scatter_start(g, b):
            di = dst_all.at[pl.ds(g * CHUNK, CHUNK)]
            pltpu.async_copy(rows[b], acc.at[di], s_sem[b], add=True)
            pltpu.async_copy(ones_v, cnt.at[di], c_sem[b], add=True)

        def scatter_wait(g, b):
            di = dst_all.at[pl.ds(g * CHUNK, CHUNK)]
            pltpu.make_async_copy(rows[b], acc.at[di], s_sem[b]).wait()
            pltpu.make_async_copy(ones_v, cnt.at[di], c_sem[b]).wait()

        gather_start(0, 0)
        gather_wait(0, 0)
        scatter_start(0, 0)
        gather_start(1, 1)

        def pair_body(p, _):
            g = 2 * p + 1
            gather_wait(g, 1)
            scatter_start(g, 1)
            scatter_wait(g - 1, 0)
            gather_start(g + 1, 0)
            gather_wait(g + 1, 0)
            scatter_start(g + 1, 0)
            scatter_wait(g, 1)
            gather_start(g + 2, 1)
            return 0
        lax.fori_loop(0, (N_CHUNKS - 2) // 2, pair_body, 0)

        gather_wait(N_CHUNKS - 1, 1)
        scatter_start(N_CHUNKS - 1, 1)
        scatter_wait(N_CHUNKS - 2, 0)
        scatter_wait(N_CHUNKS - 1, 1)
        plsc.subcore_barrier()

        def finish(rbase, rws):
            pltpu.sync_copy(acc.at[pl.ds(rbase, rws)],
                            rows_v0.at[pl.ds(0, rws)])
            pltpu.sync_copy(cnt.at[pl.ds(rbase, rws)],
                            cnt_v.at[pl.ds(0, rws)])
            pltpu.sync_copy(r_hbm.at[pl.ds(rbase, rws)],
                            rows_v1.at[pl.ds(0, rws)])

            def grp(j, _):
                c16 = cnt_v[pl.ds(j * 16, 16)]
                ones_v[pl.ds(j * 16, 16)] = 1.0 / jnp.maximum(c16, 1.0)
                return 0
            lax.fori_loop(0, rws // 16, grp, 0)

            def rowb(i, _):
                s = ones_v[pl.ds(i, 16)][0]
                rows_v0[i, pl.ds(0, 16)] = (rows_v0[i, pl.ds(0, 16)] * s
                                            + rows_v1[i, pl.ds(0, 16)])
                return 0
            lax.fori_loop(0, rws, rowb, 0)
            pltpu.sync_copy(rows_v0.at[pl.ds(0, rws)],
                            out_hbm.at[pl.ds(rbase, rws)])

        @pl.when(sid < NS - 1)
        def _():
            finish(sid * ROWS_PER_TILE, ROWS_PER_TILE)

        @pl.when(sid == NS - 1)
        def _():
            finish((NS - 1) * ROWS_PER_TILE, LAST_ROWS)

    @pl.when(cid == 0)
    def _():
        process(h1_hbm, e1_hbm, r1_hbm, out1_hbm)

    @pl.when(cid == 1)
    def _():
        process(h2_hbm, e2_hbm, r2_hbm, out2_hbm)


def kernel(x_account, x_transaction, edge_index_initiates, edge_index_receives,
           W_l1, b_l1, W_r1, W_l2, b_l2, W_r2):
    e1 = (edge_index_initiates if edge_index_initiates.dtype == jnp.int32
          else edge_index_initiates.astype(jnp.int32))
    e2 = (edge_index_receives if edge_index_receives.dtype == jnp.int32
          else edge_index_receives.astype(jnp.int32))

    stage1 = pl.pallas_call(
        _stage1_body,
        grid=(N_NODE // BLK,),
        in_specs=[
            pl.BlockSpec((BLK, D_FEAT), lambda i: (i, 0)),
            pl.BlockSpec((BLK, D_FEAT), lambda i: (i, 0)),
            pl.BlockSpec((D_FEAT, D_OUT), lambda i: (0, 0)),
            pl.BlockSpec((1, D_OUT), lambda i: (0, 0)),
            pl.BlockSpec((D_FEAT, D_OUT), lambda i: (0, 0)),
            pl.BlockSpec((D_FEAT, D_OUT), lambda i: (0, 0)),
            pl.BlockSpec((1, D_OUT), lambda i: (0, 0)),
            pl.BlockSpec((D_FEAT, D_OUT), lambda i: (0, 0)),
        ],
        out_specs=[
            pl.BlockSpec((BLK, D_OUT), lambda i: (i, 0)),
            pl.BlockSpec((BLK, D_OUT), lambda i: (i, 0)),
            pl.BlockSpec((BLK, D_OUT), lambda i: (i, 0)),
            pl.BlockSpec((BLK, D_OUT), lambda i: (i, 0)),
        ],
        out_shape=[
            jax.ShapeDtypeStruct((N_NODE, D_OUT), jnp.float32),
            jax.ShapeDtypeStruct((N_NODE, D_OUT), jnp.float32),
            jax.ShapeDtypeStruct((N_NODE, D_OUT), jnp.float32),
            jax.ShapeDtypeStruct((N_NODE, D_OUT), jnp.float32),
        ],
    )
    h1, h2, r1, r2 = stage1(x_account, x_transaction,
                            W_l1, b_l1.reshape(1, D_OUT), W_r1,
                            W_l2, b_l2.reshape(1, D_OUT), W_r2)

    seg = pl.kernel(
        _segment_kernel,
        out_type=[
            jax.ShapeDtypeStruct((N_NODE, D_OUT), jnp.float32),
            jax.ShapeDtypeStruct((N_NODE, D_OUT), jnp.float32),
        ],
        mesh=plsc.VectorSubcoreMesh(core_axis_name="c", subcore_axis_name="s",
                                    num_cores=NC, num_subcores=NS),
        compiler_params=pltpu.CompilerParams(use_tc_tiling_on_sc=False,
                                             skip_device_barrier=True),
        scratch_types=[
            pltpu.VMEM_SHARED((N_NODE, D_OUT), jnp.float32),
            pltpu.VMEM_SHARED((N_NODE,), jnp.float32),
            pltpu.VMEM_SHARED((N_NODE, D_OUT), jnp.float32),
            pltpu.VMEM((EDGES_PER_TILE,), jnp.int32),
            pltpu.VMEM((EDGES_PER_TILE,), jnp.int32),
            pltpu.VMEM((CHUNK, D_OUT), jnp.float32),
            pltpu.VMEM((CHUNK, D_OUT), jnp.float32),
            pltpu.VMEM((CHUNK,), jnp.float32),
            pltpu.VMEM((LAST_ROWS,), jnp.float32),
            pltpu.SemaphoreType.DMA,
            pltpu.SemaphoreType.DMA,
            pltpu.SemaphoreType.DMA,
            pltpu.SemaphoreType.DMA,
            pltpu.SemaphoreType.DMA,
            pltpu.SemaphoreType.DMA,
        ],
    )
    out_transaction, out_account = seg(h1, e1, h2, e2, r1, r2)
    return (out_transaction, out_account)
